# baseline (speedup 1.0000x reference)
import jax
import jax.numpy as jnp
from jax import lax
from jax.experimental import pallas as pl
from jax.experimental.pallas import tpu as pltpu
from jax.experimental.pallas import tpu_sc as plsc

N = 10000
E = 320000
D = 128

NC = 2
NS = 16
NW = NC * NS

CH = 128
SCH = 16
NSUP = 5
NCHUNK = SCH * NSUP
EPW = NCHUNK * CH
E_PAD = NW * EPW

RPT = 624
RPT_LAST = 656
N_PAD = (NS - 1) * RPT + RPT_LAST

_SELU_ALPHA = 1.6732632423543772
_SELU_SCALE = 1.0507009873554805


def _mm_body(x_ref, wt_ref, b_ref, o_ref):
    o_ref[...] = (
        jnp.dot(x_ref[...], wt_ref[...], preferred_element_type=jnp.float32)
        + b_ref[...]
    )


def _matmul(x, wt, b2):
    blk = 2000
    grid = (N // blk,)
    return pl.pallas_call(
        _mm_body,
        grid=grid,
        in_specs=[
            pl.BlockSpec((blk, D), lambda i: (i, 0)),
            pl.BlockSpec((D, D), lambda i: (0, 0)),
            pl.BlockSpec((1, D), lambda i: (0, 0)),
        ],
        out_specs=pl.BlockSpec((blk, D), lambda i: (i, 0)),
        out_shape=jax.ShapeDtypeStruct((N, D), jnp.float32),
    )(x, wt, b2)


def _sc_body(t_hbm, src_hbm, dst_hbm, w_hbm, out_hbm,
             acc, srcb, dstb0, dstb1, wb, rows0, rows1,
             gsem, ssem0, ssem1):
    cid = lax.axis_index("c")
    sid = lax.axis_index("s")
    wid = cid * NS + sid
    dstb = (dstb0, dstb1)
    rows = (rows0, rows1)
    ssem = (ssem0, ssem1)

    stripe0 = sid * RPT

    z16 = jnp.zeros((16,), jnp.float32)

    def _zrow(i, car):
        for jj in range(D // 16):
            rows0[i, pl.ds(jj * 16, 16)] = z16
        return car

    lax.fori_loop(0, CH, _zrow, 0)

    @pl.when(sid < NS - 1)
    def _():
        for k in range(RPT // CH):
            pltpu.sync_copy(rows0, acc.at[pl.ds(stripe0 + k * CH, CH)])
        rem = RPT % CH
        pltpu.sync_copy(
            rows0.at[pl.ds(0, rem)],
            acc.at[pl.ds(stripe0 + (RPT // CH) * CH, rem)],
        )

    @pl.when(sid == NS - 1)
    def _():
        for k in range(RPT_LAST // CH):
            pltpu.sync_copy(rows0, acc.at[pl.ds(stripe0 + k * CH, CH)])
        rem = RPT_LAST % CH
        pltpu.sync_copy(
            rows0.at[pl.ds(0, rem)],
            acc.at[pl.ds(stripe0 + (RPT_LAST // CH) * CH, rem)],
        )

    plsc.subcore_barrier()

    def _scale(j, b, wbr):
        rr = rows[b]

        def _grp(gi, carry2):
            wv16 = wbr[j, pl.ds(gi * 16, 16)]
            for e16 in range(16):
                wsp = wv16.at[jnp.full((16,), e16, jnp.int32)].get(
                    mode="promise_in_bounds"
                )
                r = gi * 16 + e16
                for jj in range(D // 16):
                    sl = pl.ds(jj * 16, 16)
                    rr[r, sl] = rr[r, sl] * wsp
            return carry2

        lax.fori_loop(0, CH // 16, _grp, 0)

    def _step(c, j, b, db, guard):
        if guard:
            @pl.when(c >= 2)
            def _():
                pltpu.make_async_copy(
                    rows[b], acc.at[db.at[j]], ssem[b]
                ).wait()
        else:
            pltpu.make_async_copy(rows[b], acc.at[db.at[j]], ssem[b]).wait()

        pltpu.async_copy(t_hbm.at[srcb.at[j]], rows[b], gsem).wait()
        _scale(j, b, wb)
        pltpu.async_copy(rows[b], acc.at[db.at[j]], ssem[b], add=True)

    for s in range(NSUP):
        db = dstb[s % 2]
        pltpu.sync_copy(src_hbm.at[wid, s], srcb)
        pltpu.sync_copy(dst_hbm.at[wid, s], db)
        pltpu.sync_copy(w_hbm.at[wid, s], wb)

        def _pair(j2, car, s=s, db=db):
            j = j2 * 2
            c = s * SCH + j
            _step(c, j, 0, db, s == 0)
            _step(c + 1, j + 1, 1, db, s == 0)
            return car

        lax.fori_loop(0, SCH // 2, _pair, 0)

    pltpu.make_async_copy(rows[0], acc.at[dstb0.at[0]], ssem[0]).wait()
    pltpu.make_async_copy(rows[1], acc.at[dstb0.at[0]], ssem[1]).wait()

    plsc.subcore_barrier()

    out_base = cid * N_PAD + stripe0

    @pl.when(sid < NS - 1)
    def _():
        for k in range(RPT // CH):
            pltpu.sync_copy(acc.at[pl.ds(stripe0 + k * CH, CH)], rows0)
            pltpu.sync_copy(rows0, out_hbm.at[pl.ds(out_base + k * CH, CH)])
        rem = RPT % CH
        off = (RPT // CH) * CH
        pltpu.sync_copy(
            acc.at[pl.ds(stripe0 + off, rem)], rows0.at[pl.ds(0, rem)]
        )
        pltpu.sync_copy(
            rows0.at[pl.ds(0, rem)], out_hbm.at[pl.ds(out_base + off, rem)]
        )

    @pl.when(sid == NS - 1)
    def _():
        for k in range(RPT_LAST // CH):
            pltpu.sync_copy(acc.at[pl.ds(stripe0 + k * CH, CH)], rows0)
            pltpu.sync_copy(rows0, out_hbm.at[pl.ds(out_base + k * CH, CH)])
        rem = RPT_LAST % CH
        off = (RPT_LAST // CH) * CH
        pltpu.sync_copy(
            acc.at[pl.ds(stripe0 + off, rem)], rows0.at[pl.ds(0, rem)]
        )
        pltpu.sync_copy(
            rows0.at[pl.ds(0, rem)], out_hbm.at[pl.ds(out_base + off, rem)]
        )


def _scatter(transformed, src4, dst4, w4):
    mesh = plsc.VectorSubcoreMesh(core_axis_name="c", subcore_axis_name="s")
    return pl.kernel(
        _sc_body,
        out_type=jax.ShapeDtypeStruct((NC * N_PAD, D), jnp.float32),
        mesh=mesh,
        scratch_types=[
            pltpu.VMEM_SHARED((N_PAD, D), jnp.float32),
            pltpu.VMEM((SCH, CH), jnp.int32),
            pltpu.VMEM((SCH, CH), jnp.int32),
            pltpu.VMEM((SCH, CH), jnp.int32),
            pltpu.VMEM((SCH, CH), jnp.float32),
            pltpu.VMEM((CH, D), jnp.float32),
            pltpu.VMEM((CH, D), jnp.float32),
            pltpu.SemaphoreType.DMA,
            pltpu.SemaphoreType.DMA,
            pltpu.SemaphoreType.DMA,
        ],
    )(transformed, src4, dst4, w4)


def _fin_body(t_ref, p0_ref, p1_ref, skip_ref, o_ref):
    z = skip_ref[...] * t_ref[...] + p0_ref[...] + p1_ref[...]
    o_ref[...] = _SELU_SCALE * jnp.where(
        z > 0, z, _SELU_ALPHA * (jnp.exp(z) - 1.0)
    )


def _finish(transformed, p0, p1, skip2):
    blk = 2000
    grid = (N // blk,)
    bs = pl.BlockSpec((blk, D), lambda i: (i, 0))
    return pl.pallas_call(
        _fin_body,
        grid=grid,
        in_specs=[bs, bs, bs, pl.BlockSpec((1, D), lambda i: (0, 0))],
        out_specs=bs,
        out_shape=jax.ShapeDtypeStruct((N, D), jnp.float32),
    )(transformed, p0, p1, skip2)


def _pad_edges(v, fill):
    pad = jnp.full((E_PAD - E,), fill, v.dtype)
    return jnp.concatenate([v, pad]).reshape(NW, NSUP, SCH, CH)


@jax.jit
def kernel(x, edge_index, edge_weight, W, b, skip_weight):
    transformed = _matmul(x, W.T, b.reshape(1, D))
    src4 = _pad_edges(edge_index[1].astype(jnp.int32), 0)
    dst4 = _pad_edges(edge_index[0].astype(jnp.int32), N)
    w4 = _pad_edges(edge_weight, 0.0)
    partials = _scatter(transformed, src4, dst4, w4)
    return _finish(
        transformed,
        partials[:N],
        partials[N_PAD:N_PAD + N],
        skip_weight.reshape(1, D),
    )


# speedup vs baseline: 1.9306x; 1.9306x over previous
import functools

import jax
import jax.numpy as jnp
from jax import lax
from jax.experimental import pallas as pl
from jax.experimental.pallas import tpu as pltpu
from jax.experimental.pallas import tpu_sc as plsc

N = 10000
E = 320000
D = 128

NC = 2
NS = 16
NW = NC * NS

CH = 80
EPW = E // NW
NCHUNK = EPW // CH
RPT = 632
N_PAD = NS * RPT

_SELU_ALPHA = 1.6732632423543772
_SELU_SCALE = 1.0507009873554805


def _mm_body(x_ref, wt_ref, b_ref, o_ref):
    o_ref[...] = (
        jnp.dot(x_ref[...], wt_ref[...], preferred_element_type=jnp.float32)
        + b_ref[...]
    )


def _matmul(x, wt, b2):
    blk = 2000
    grid = (N // blk,)
    return pl.pallas_call(
        _mm_body,
        grid=grid,
        in_specs=[
            pl.BlockSpec((blk, D), lambda i: (i, 0)),
            pl.BlockSpec((D, D), lambda i: (0, 0)),
            pl.BlockSpec((1, D), lambda i: (0, 0)),
        ],
        out_specs=pl.BlockSpec((blk, D), lambda i: (i, 0)),
        out_shape=jax.ShapeDtypeStruct((N, D), jnp.float32),
    )(x, wt, b2)


def _sc_body(t_hbm, src_hbm, dst_hbm, w_hbm, z_hbm, out_hbm,
             acc, srcbuf, dstbuf, wbuf, rows, sem):
    cid = lax.axis_index("c")
    sid = lax.axis_index("s")
    wid = cid * NS + sid

    pltpu.sync_copy(src_hbm.at[pl.ds(wid * EPW, EPW)], srcbuf)
    pltpu.sync_copy(dst_hbm.at[wid], dstbuf)
    pltpu.sync_copy(w_hbm.at[pl.ds(wid * EPW, EPW)], wbuf)

    pltpu.sync_copy(z_hbm, acc.at[pl.ds(sid * RPT, RPT)])
    plsc.subcore_barrier()

    def _chunk(c, carry):
        pltpu.async_copy(
            t_hbm.at[srcbuf.at[pl.ds(c * CH, CH)]], rows, sem
        ).wait()

        for g in range(CH // 16):
            wv16 = wbuf[pl.ds(c * CH + g * 16, 16)]
            for e16 in range(16):
                wsp = wv16.at[jnp.full((16,), e16, jnp.int32)].get(
                    mode="promise_in_bounds"
                )
                r = g * 16 + e16
                for j in range(D // 16):
                    sl = pl.ds(j * 16, 16)
                    rows[r, sl] = rows[r, sl] * wsp
        pltpu.sync_copy(rows, acc.at[dstbuf.at[c]], add=True)
        return carry

    lax.fori_loop(0, NCHUNK, _chunk, 0)

    plsc.subcore_barrier()
    pltpu.sync_copy(
        acc.at[pl.ds(sid * RPT, RPT)],
        out_hbm.at[pl.ds(cid * N_PAD + sid * RPT, RPT)],
    )


def _scatter(transformed, src, dst3d, w, zrows):
    mesh = plsc.VectorSubcoreMesh(core_axis_name="c", subcore_axis_name="s")
    return pl.kernel(
        _sc_body,
        out_type=jax.ShapeDtypeStruct((NC * N_PAD, D), jnp.float32),
        mesh=mesh,
        scratch_types=[
            pltpu.VMEM_SHARED((N_PAD, D), jnp.float32),
            pltpu.VMEM((EPW,), jnp.int32),
            pltpu.VMEM((NCHUNK, CH), jnp.int32),
            pltpu.VMEM((EPW,), jnp.float32),
            pltpu.VMEM((CH, D), jnp.float32),
            pltpu.SemaphoreType.DMA,
        ],
    )(transformed, src, dst3d, w, zrows)


def _fin_body(t_ref, p0_ref, p1_ref, skip_ref, o_ref):
    z = skip_ref[...] * t_ref[...] + p0_ref[...] + p1_ref[...]
    o_ref[...] = _SELU_SCALE * jnp.where(
        z > 0, z, _SELU_ALPHA * (jnp.exp(z) - 1.0)
    )


def _finish(transformed, p0, p1, skip2):
    blk = 2000
    grid = (N // blk,)
    bs = pl.BlockSpec((blk, D), lambda i: (i, 0))
    return pl.pallas_call(
        _fin_body,
        grid=grid,
        in_specs=[bs, bs, bs, pl.BlockSpec((1, D), lambda i: (0, 0))],
        out_specs=bs,
        out_shape=jax.ShapeDtypeStruct((N, D), jnp.float32),
    )(transformed, p0, p1, skip2)


@jax.jit
def kernel(x, edge_index, edge_weight, W, b, skip_weight):
    transformed = _matmul(x, W.T, b.reshape(1, D))
    src = edge_index[1].astype(jnp.int32)
    dst3d = edge_index[0].astype(jnp.int32).reshape(NW, NCHUNK, CH)
    zrows = jnp.zeros((RPT, D), jnp.float32)
    partials = _scatter(transformed, src, dst3d, edge_weight, zrows)
    return _finish(
        transformed,
        partials[:N],
        partials[N_PAD:N_PAD + N],
        skip_weight.reshape(1, D),
    )


# speedup vs baseline: 2.3033x; 1.1930x over previous
import functools

import jax
import jax.numpy as jnp
from jax import lax
from jax.experimental import pallas as pl
from jax.experimental.pallas import tpu as pltpu
from jax.experimental.pallas import tpu_sc as plsc

N = 10000
E = 320000
D = 128

NC = 2
NS = 16
NW = NC * NS

CH = 80
EPW = E // NW
NCHUNK = EPW // CH
RPT = 624
RPT_LAST = 640
N_PAD = (NS - 1) * RPT + RPT_LAST

_SELU_ALPHA = 1.6732632423543772
_SELU_SCALE = 1.0507009873554805


def _mm_body(x_ref, wt_ref, b_ref, o_ref):
    o_ref[...] = (
        jnp.dot(x_ref[...], wt_ref[...], preferred_element_type=jnp.float32)
        + b_ref[...]
    )


def _matmul(x, wt, b2):
    blk = 2000
    grid = (N // blk,)
    return pl.pallas_call(
        _mm_body,
        grid=grid,
        in_specs=[
            pl.BlockSpec((blk, D), lambda i: (i, 0)),
            pl.BlockSpec((D, D), lambda i: (0, 0)),
            pl.BlockSpec((1, D), lambda i: (0, 0)),
        ],
        out_specs=pl.BlockSpec((blk, D), lambda i: (i, 0)),
        out_shape=jax.ShapeDtypeStruct((N, D), jnp.float32),
    )(x, wt, b2)


def _sc_body(t_hbm, src_hbm, dst_hbm, w_hbm, out_hbm,
             acc, srcbuf, dstbuf, wbuf, rows0, rows1, gsem, ssem0, ssem1):
    rows_ring = (rows0, rows1)
    ssem = (ssem0, ssem1)
    cid = lax.axis_index("c")
    sid = lax.axis_index("s")
    wid = cid * NS + sid

    pltpu.sync_copy(src_hbm.at[pl.ds(wid * EPW, EPW)], srcbuf)
    pltpu.sync_copy(dst_hbm.at[pl.ds(wid * EPW, EPW)], dstbuf)
    pltpu.sync_copy(w_hbm.at[pl.ds(wid * EPW, EPW)], wbuf)

    z16 = jnp.zeros((16,), jnp.float32)

    def _zrow(i, car):
        for jj in range(D // 16):
            rows0[i, pl.ds(jj * 16, 16)] = z16
        return car

    lax.fori_loop(0, CH, _zrow, 0)

    def _fill(nrows):
        for k in range(nrows // CH):
            pltpu.sync_copy(rows0, acc.at[pl.ds(sid * RPT + k * CH, CH)])
        rem = nrows % CH
        if rem:
            pltpu.sync_copy(
                rows0.at[pl.ds(0, rem)],
                acc.at[pl.ds(sid * RPT + (nrows // CH) * CH, rem)],
            )

    @pl.when(sid < NS - 1)
    def _():
        _fill(RPT)

    @pl.when(sid == NS - 1)
    def _():
        _fill(RPT_LAST)

    plsc.subcore_barrier()

    def _step(c, b, guard):
        rows = rows_ring[b]
        if guard:
            @pl.when(c >= 2)
            def _():
                pltpu.make_async_copy(
                    rows, acc.at[dstbuf.at[pl.ds(c * CH, CH)]], ssem[b]
                ).wait()
        else:
            pltpu.make_async_copy(rows, acc.at[dstbuf.at[pl.ds(c * CH, CH)]], ssem[b]).wait()

        pltpu.async_copy(
            t_hbm.at[srcbuf.at[pl.ds(c * CH, CH)]], rows, gsem
        ).wait()

        for g in range(CH // 16):
            wv16 = wbuf[pl.ds(c * CH + g * 16, 16)]
            for e16 in range(16):
                wsp = wv16.at[jnp.full((16,), e16, jnp.int32)].get(
                    mode="promise_in_bounds"
                )
                r = g * 16 + e16
                for j in range(D // 16):
                    sl = pl.ds(j * 16, 16)
                    rows[r, sl] = rows[r, sl] * wsp

        pltpu.async_copy(rows, acc.at[dstbuf.at[pl.ds(c * CH, CH)]], ssem[b], add=True)

    _step(0, 0, True)
    _step(1, 1, True)

    def _pair(i, carry):
        c = 2 + i * 2
        _step(c, 0, False)
        _step(c + 1, 1, False)
        return carry

    lax.fori_loop(0, (NCHUNK - 2) // 2, _pair, 0)
    _step(NCHUNK - 1, 0, False)

    pltpu.make_async_copy(rows_ring[1], acc.at[dstbuf.at[pl.ds(0, CH)]], ssem[1]).wait()
    pltpu.make_async_copy(rows_ring[0], acc.at[dstbuf.at[pl.ds(0, CH)]], ssem[0]).wait()

    plsc.subcore_barrier()
    _obase = cid * N_PAD + sid * RPT

    def _flush(nrows):
        for k in range(nrows // CH):
            pltpu.sync_copy(acc.at[pl.ds(sid * RPT + k * CH, CH)], rows0)
            pltpu.sync_copy(rows0, out_hbm.at[pl.ds(_obase + k * CH, CH)])

    @pl.when(sid < NS - 1)
    def _():
        _flush(RPT)
        rem = RPT % CH
        off = (RPT // CH) * CH
        pltpu.sync_copy(
            acc.at[pl.ds(sid * RPT + off, rem)], rows0.at[pl.ds(0, rem)]
        )
        pltpu.sync_copy(
            rows0.at[pl.ds(0, rem)], out_hbm.at[pl.ds(_obase + off, rem)]
        )

    @pl.when(sid == NS - 1)
    def _():
        _flush(RPT_LAST)


def _scatter(transformed, src, dst, w):
    mesh = plsc.VectorSubcoreMesh(core_axis_name="c", subcore_axis_name="s")
    return pl.kernel(
        _sc_body,
        out_type=jax.ShapeDtypeStruct((NC * N_PAD, D), jnp.float32),
        mesh=mesh,
        scratch_types=[
            pltpu.VMEM_SHARED((N_PAD, D), jnp.float32),
            pltpu.VMEM((EPW,), jnp.int32),
            pltpu.VMEM((EPW,), jnp.int32),
            pltpu.VMEM((EPW,), jnp.float32),
            pltpu.VMEM((CH, D), jnp.float32),
            pltpu.VMEM((CH, D), jnp.float32),
            pltpu.SemaphoreType.DMA,
            pltpu.SemaphoreType.DMA,
            pltpu.SemaphoreType.DMA,
        ],
    )(transformed, src, dst, w)


def _fin_body(t_ref, p0_ref, p1_ref, skip_ref, o_ref):
    z = skip_ref[...] * t_ref[...] + p0_ref[...] + p1_ref[...]
    o_ref[...] = _SELU_SCALE * jnp.where(
        z > 0, z, _SELU_ALPHA * (jnp.exp(z) - 1.0)
    )


def _finish(transformed, p0, p1, skip2):
    blk = 2000
    grid = (N // blk,)
    bs = pl.BlockSpec((blk, D), lambda i: (i, 0))
    return pl.pallas_call(
        _fin_body,
        grid=grid,
        in_specs=[bs, bs, bs, pl.BlockSpec((1, D), lambda i: (0, 0))],
        out_specs=bs,
        out_shape=jax.ShapeDtypeStruct((N, D), jnp.float32),
    )(transformed, p0, p1, skip2)


@jax.jit
def kernel(x, edge_index, edge_weight, W, b, skip_weight):
    transformed = _matmul(x, W.T, b.reshape(1, D))
    src = edge_index[1].astype(jnp.int32)
    dst = edge_index[0].astype(jnp.int32)
    partials = _scatter(transformed, src, dst, edge_weight)
    return _finish(
        transformed,
        partials[:N],
        partials[N_PAD:N_PAD + N],
        skip_weight.reshape(1, D),
    )


# speedup vs baseline: 2.3676x; 1.0279x over previous
import functools

import jax
import jax.numpy as jnp
from jax import lax
from jax.experimental import pallas as pl
from jax.experimental.pallas import tpu as pltpu
from jax.experimental.pallas import tpu_sc as plsc

N = 10000
E = 320000
D = 128

NC = 2
NS = 16
NW = NC * NS

CH = 80
EPW = E // NW
NCHUNK = EPW // CH
RPT = 624
RPT_LAST = 640
N_PAD = (NS - 1) * RPT + RPT_LAST

_SELU_ALPHA = 1.6732632423543772
_SELU_SCALE = 1.0507009873554805


def _mm_body(x_ref, wt_ref, b_ref, o_ref):
    o_ref[...] = (
        jnp.dot(x_ref[...], wt_ref[...], preferred_element_type=jnp.float32)
        + b_ref[...]
    )


def _matmul(x, wt, b2):
    blk = 2000
    grid = (N // blk,)
    return pl.pallas_call(
        _mm_body,
        grid=grid,
        in_specs=[
            pl.BlockSpec((blk, D), lambda i: (i, 0)),
            pl.BlockSpec((D, D), lambda i: (0, 0)),
            pl.BlockSpec((1, D), lambda i: (0, 0)),
        ],
        out_specs=pl.BlockSpec((blk, D), lambda i: (i, 0)),
        out_shape=jax.ShapeDtypeStruct((N, D), jnp.float32),
    )(x, wt, b2)


def _sc_body(t_hbm, src_hbm, dst_hbm, w_hbm, out_hbm,
             acc, srcbuf, dstbuf, wbuf, rows0, rows1, gsem, ssem0, ssem1):
    rows_ring = (rows0, rows1)
    ssem = (ssem0, ssem1)
    cid = lax.axis_index("c")
    sid = lax.axis_index("s")
    wid = cid * NS + sid

    pltpu.sync_copy(src_hbm.at[pl.ds(wid * EPW, EPW)], srcbuf)
    pltpu.sync_copy(dst_hbm.at[pl.ds(wid * EPW, EPW)], dstbuf)
    pltpu.sync_copy(w_hbm.at[pl.ds(wid * EPW, EPW)], wbuf)

    z16 = jnp.zeros((16,), jnp.float32)

    def _zrow(i, car):
        for jj in range(D // 16):
            rows0[i, pl.ds(jj * 16, 16)] = z16
        return car

    lax.fori_loop(0, CH, _zrow, 0)

    def _fill(nrows):
        for k in range(nrows // CH):
            pltpu.sync_copy(rows0, acc.at[pl.ds(sid * RPT + k * CH, CH)])
        rem = nrows % CH
        if rem:
            pltpu.sync_copy(
                rows0.at[pl.ds(0, rem)],
                acc.at[pl.ds(sid * RPT + (nrows // CH) * CH, rem)],
            )

    @pl.when(sid < NS - 1)
    def _():
        _fill(RPT)

    @pl.when(sid == NS - 1)
    def _():
        _fill(RPT_LAST)

    plsc.subcore_barrier()

    def _step(c, b, guard):
        rows = rows_ring[b]
        if guard:
            @pl.when(c >= 2)
            def _():
                pltpu.make_async_copy(
                    rows, acc.at[dstbuf.at[pl.ds(c * CH, CH)]], ssem[b]
                ).wait()
        else:
            pltpu.make_async_copy(rows, acc.at[dstbuf.at[pl.ds(c * CH, CH)]], ssem[b]).wait()

        pltpu.async_copy(
            t_hbm.at[srcbuf.at[pl.ds(c * CH, CH)]], rows, gsem
        ).wait()

        for g in range(CH // 16):
            wv16 = wbuf[pl.ds(c * CH + g * 16, 16)]
            for e16 in range(16):
                wsp = wv16.at[jnp.full((16,), e16, jnp.int32)].get(
                    mode="promise_in_bounds"
                )
                r = g * 16 + e16
                for j in range(D // 16):
                    sl = pl.ds(j * 16, 16)
                    rows[r, sl] = rows[r, sl] * wsp

        pltpu.async_copy(rows, acc.at[dstbuf.at[pl.ds(c * CH, CH)]], ssem[b], add=True)

    _step(0, 0, True)
    _step(1, 1, True)

    def _pair(i, carry):
        c = 2 + i * 2
        _step(c, 0, False)
        _step(c + 1, 1, False)
        return carry

    lax.fori_loop(0, (NCHUNK - 2) // 2, _pair, 0)
    _step(NCHUNK - 1, 0, False)

    pltpu.make_async_copy(rows_ring[1], acc.at[dstbuf.at[pl.ds(0, CH)]], ssem[1]).wait()
    pltpu.make_async_copy(rows_ring[0], acc.at[dstbuf.at[pl.ds(0, CH)]], ssem[0]).wait()

    plsc.subcore_barrier()
    _obase = cid * N_PAD + sid * RPT

    def _flush(nrows):
        for k in range(nrows // CH):
            pltpu.sync_copy(acc.at[pl.ds(sid * RPT + k * CH, CH)], rows0)
            pltpu.sync_copy(rows0, out_hbm.at[pl.ds(_obase + k * CH, CH)])

    @pl.when(sid < NS - 1)
    def _():
        _flush(RPT)
        rem = RPT % CH
        off = (RPT // CH) * CH
        pltpu.sync_copy(
            acc.at[pl.ds(sid * RPT + off, rem)], rows0.at[pl.ds(0, rem)]
        )
        pltpu.sync_copy(
            rows0.at[pl.ds(0, rem)], out_hbm.at[pl.ds(_obase + off, rem)]
        )

    @pl.when(sid == NS - 1)
    def _():
        _flush(RPT_LAST)


def _scatter(transformed, src, dst, w):
    mesh = plsc.VectorSubcoreMesh(core_axis_name="c", subcore_axis_name="s")
    return pl.kernel(
        _sc_body,
        out_type=jax.ShapeDtypeStruct((NC * N_PAD, D), jnp.float32),
        mesh=mesh,
        scratch_types=[
            pltpu.VMEM_SHARED((N_PAD, D), jnp.float32),
            pltpu.VMEM((EPW,), jnp.int32),
            pltpu.VMEM((EPW,), jnp.int32),
            pltpu.VMEM((EPW,), jnp.float32),
            pltpu.VMEM((CH, D), jnp.float32),
            pltpu.VMEM((CH, D), jnp.float32),
            pltpu.SemaphoreType.DMA,
            pltpu.SemaphoreType.DMA,
            pltpu.SemaphoreType.DMA,
        ],
    )(transformed, src, dst, w)


def _fin_body(t_ref, p0_ref, p1_ref, skip_ref, o_ref):
    z = skip_ref[...] * t_ref[...] + p0_ref[...] + p1_ref[...]
    o_ref[...] = _SELU_SCALE * jnp.where(
        z > 0, z, _SELU_ALPHA * (jnp.exp(z) - 1.0)
    )


def _finish(transformed, partials, skip2):
    blk = 2000
    grid = (N // blk,)
    bs = pl.BlockSpec((blk, D), lambda i: (i, 0))
    bs_p0 = pl.BlockSpec((blk, D), lambda i: (i, 0))
    bs_p1 = pl.BlockSpec((blk, D), lambda i: (i + N_PAD // blk, 0))
    return pl.pallas_call(
        _fin_body,
        grid=grid,
        in_specs=[bs, bs_p0, bs_p1, pl.BlockSpec((1, D), lambda i: (0, 0))],
        out_specs=bs,
        out_shape=jax.ShapeDtypeStruct((N, D), jnp.float32),
    )(transformed, partials, partials, skip2)


@jax.jit
def kernel(x, edge_index, edge_weight, W, b, skip_weight):
    transformed = _matmul(x, W.T, b.reshape(1, D))
    src = edge_index[1].astype(jnp.int32)
    dst = edge_index[0].astype(jnp.int32)
    partials = _scatter(transformed, src, dst, edge_weight)
    return _finish(transformed, partials, skip_weight.reshape(1, D))
